# baseline (device time: 92271 ns/iter reference)
import functools

import jax
import jax.numpy as jnp
from jax import lax
from jax.experimental import pallas as pl
from jax.experimental.pallas import tpu as pltpu

B = 16
H = 16
D = 64
BS = 16
NB = 128
LP = 128
NK = LP * BS
SCALE = D ** -0.5
NEG = -1e30

PACK_ROWS = H * B + B


def _body(q_ref, k_ref, v_ref, bt_ref, lens_ref, out_ref,
          send_buf, recv_buf, send_sem, recv_sem, exit_sem):
    my_x = lax.axis_index("x")
    my_y = lax.axis_index("y")
    my_z = lax.axis_index("z")
    nbr = (1 - my_x, my_y, my_z)

    slot = lax.broadcasted_iota(jnp.int32, (B, NB), 1)
    valid = slot < lens_ref[:, :]
    bt = jnp.where(valid, bt_ref[:, :], -1)
    page0 = my_x * LP
    pid = page0 + lax.broadcasted_iota(jnp.int32, (LP, 1), 0)
    counts = []
    for i in range(B):
        bti = bt[i, :].reshape(1, NB)
        eq = (bti == pid).astype(jnp.float32)
        counts.append(jnp.sum(eq, axis=1))
    C = jnp.stack(counts, axis=0)
    Ck = jnp.broadcast_to(C[:, :, None], (B, LP, BS)).reshape(B, NK)
    sel = Ck > 0.0

    q = q_ref[:, 0, :, :]
    m_list, l_list, acc_list = [], [], []
    for h in range(H):
        qh = q[:, h, :].astype(jnp.bfloat16)
        kh = k_ref[:, :, h, :].reshape(NK, D).astype(jnp.bfloat16)
        vh = v_ref[:, :, h, :].reshape(NK, D).astype(jnp.bfloat16)
        s = lax.dot_general(
            qh, kh, (((1,), (1,)), ((), ())),
            preferred_element_type=jnp.float32,
        ) * SCALE
        sm = jnp.where(sel, s, NEG)
        mh = jnp.max(sm, axis=1, keepdims=True)
        p = jnp.where(sel, Ck * jnp.exp(s - mh), 0.0)
        lh = jnp.sum(p, axis=1, keepdims=True)
        acc = lax.dot_general(
            p.astype(jnp.bfloat16), vh, (((1,), (0,)), ((), ())),
            preferred_element_type=jnp.float32,
        )
        send_buf[pl.ds(h * B, B), :] = acc
        m_list.append(mh)
        l_list.append(lh)
        acc_list.append(acc)
    m_a = jnp.concatenate(m_list, axis=1)
    l_a = jnp.concatenate(l_list, axis=1)
    send_buf[pl.ds(H * B, B), 0:H] = m_a
    send_buf[pl.ds(H * B, B), H:2 * H] = l_a

    rdma = pltpu.make_async_remote_copy(
        src_ref=send_buf,
        dst_ref=recv_buf,
        send_sem=send_sem,
        recv_sem=recv_sem,
        device_id=nbr,
        device_id_type=pl.DeviceIdType.MESH,
    )
    rdma.start()
    rdma.wait()

    m_b = recv_buf[pl.ds(H * B, B), 0:H]
    l_b = recv_buf[pl.ds(H * B, B), H:2 * H]
    m = jnp.maximum(m_a, m_b)
    alpha = jnp.exp(m_a - m)
    beta = jnp.exp(m_b - m)
    lsum = alpha * l_a + beta * l_b
    for h in range(H):
        acc_b = recv_buf[pl.ds(h * B, B), :]
        num = alpha[:, h][:, None] * acc_list[h] + beta[:, h][:, None] * acc_b
        out_ref[:, 0, h, :] = num / lsum[:, h][:, None]

    pl.semaphore_signal(
        exit_sem, inc=1, device_id=nbr, device_id_type=pl.DeviceIdType.MESH
    )
    pl.semaphore_wait(exit_sem, 1)


def kernel(Q, K, V, bt, lens):
    lens2 = lens.reshape(B, 1)
    return pl.pallas_call(
        _body,
        out_shape=jax.ShapeDtypeStruct((B, 1, H, D), jnp.float32),
        in_specs=[
            pl.BlockSpec(memory_space=pltpu.VMEM),
            pl.BlockSpec(memory_space=pltpu.VMEM),
            pl.BlockSpec(memory_space=pltpu.VMEM),
            pl.BlockSpec(memory_space=pltpu.VMEM),
            pl.BlockSpec(memory_space=pltpu.VMEM),
        ],
        out_specs=pl.BlockSpec(memory_space=pltpu.VMEM),
        scratch_shapes=[
            pltpu.VMEM((PACK_ROWS, D), jnp.float32),
            pltpu.VMEM((PACK_ROWS, D), jnp.float32),
            pltpu.SemaphoreType.DMA,
            pltpu.SemaphoreType.DMA,
            pltpu.SemaphoreType.REGULAR,
        ],
    )(Q, K, V, bt, lens2)


# device time: 59746 ns/iter; 1.5444x vs baseline; 1.5444x over previous
import jax
import jax.numpy as jnp
from jax import lax
from jax.experimental import pallas as pl
from jax.experimental.pallas import tpu as pltpu

B = 16
H = 16
D = 64
BS = 16
NB = 128
LP = 128
NK = LP * BS
SCALE = D ** -0.5
NEG = -1e30


def _body(q_ref, k_ref, v_ref, bt_ref, lens_ref, out_ref,
          kbuf, vbuf, send_acc, recv_acc, send_ml, recv_ml,
          kdma_sem, vdma_sem,
          acc_send_sem, acc_recv_sem, ml_send_sem, ml_recv_sem,
          exit_sem):
    my_x = lax.axis_index("x")
    my_y = lax.axis_index("y")
    my_z = lax.axis_index("z")
    nbr = (1 - my_x, my_y, my_z)

    def head_dma(h, slot):
        k_cp = pltpu.make_async_copy(
            k_ref.at[:, :, h, :], kbuf.at[slot], kdma_sem.at[slot])
        v_cp = pltpu.make_async_copy(
            v_ref.at[:, :, h, :], vbuf.at[slot], vdma_sem.at[slot])
        k_cp.start()
        v_cp.start()
        return k_cp, v_cp

    dma0 = head_dma(0, 0)

    barrier_sem = pltpu.get_barrier_semaphore()
    pl.semaphore_signal(
        barrier_sem, inc=1, device_id=nbr,
        device_id_type=pl.DeviceIdType.MESH,
    )
    pl.semaphore_wait(barrier_sem, 1)

    slot_iota = lax.broadcasted_iota(jnp.int32, (B, NB), 1)
    valid = slot_iota < lens_ref[:, :]
    bt = jnp.where(valid, bt_ref[:, :], -1)
    page0 = my_x * LP
    pid = page0 + lax.broadcasted_iota(jnp.int32, (LP, 1), 0)
    counts = []
    for i in range(B):
        bti = bt[i, :].reshape(1, NB)
        eq = (bti == pid).astype(jnp.float32)
        counts.append(jnp.sum(eq, axis=1))
    C = jnp.stack(counts, axis=0)
    Ck = jnp.broadcast_to(C[:, :, None], (B, LP, BS)).reshape(B, NK)
    sel = Ck > 0.0

    q = q_ref[:, 0, :, :]
    m_list, l_list, acc_list, rdmas = [], [], [], []
    dma = dma0
    for h in range(H):
        if h + 1 < H:
            next_dma = head_dma(h + 1, (h + 1) % 2)
        dma[0].wait()
        dma[1].wait()
        cur = h % 2
        qh = q[:, h, :].astype(jnp.bfloat16)
        kh = kbuf[cur].reshape(NK, D).astype(jnp.bfloat16)
        vh = vbuf[cur].reshape(NK, D).astype(jnp.bfloat16)
        s = lax.dot_general(
            qh, kh, (((1,), (1,)), ((), ())),
            preferred_element_type=jnp.float32,
        ) * SCALE
        sm = jnp.where(sel, s, NEG)
        mh = jnp.max(sm, axis=1, keepdims=True)
        p = jnp.where(sel, Ck * jnp.exp(s - mh), 0.0)
        lh = jnp.sum(p, axis=1, keepdims=True)
        acc = lax.dot_general(
            p.astype(jnp.bfloat16), vh, (((1,), (0,)), ((), ())),
            preferred_element_type=jnp.float32,
        )
        send_acc[h] = acc
        rdma = pltpu.make_async_remote_copy(
            src_ref=send_acc.at[h],
            dst_ref=recv_acc.at[h],
            send_sem=acc_send_sem.at[h],
            recv_sem=acc_recv_sem.at[h],
            device_id=nbr,
            device_id_type=pl.DeviceIdType.MESH,
        )
        rdma.start()
        rdmas.append(rdma)
        m_list.append(mh)
        l_list.append(lh)
        acc_list.append(acc)
        if h + 1 < H:
            dma = next_dma
    m_a = jnp.concatenate(m_list, axis=1)
    l_a = jnp.concatenate(l_list, axis=1)
    send_ml[:, 0:H] = m_a
    send_ml[:, H:2 * H] = l_a
    ml_rdma = pltpu.make_async_remote_copy(
        src_ref=send_ml,
        dst_ref=recv_ml,
        send_sem=ml_send_sem,
        recv_sem=ml_recv_sem,
        device_id=nbr,
        device_id_type=pl.DeviceIdType.MESH,
    )
    ml_rdma.start()

    ml_rdma.wait()
    m_b = recv_ml[:, 0:H]
    l_b = recv_ml[:, H:2 * H]
    m = jnp.maximum(m_a, m_b)
    alpha = jnp.exp(m_a - m)
    beta = jnp.exp(m_b - m)
    lsum = alpha * l_a + beta * l_b
    for h in range(H):
        rdmas[h].wait()
        acc_b = recv_acc[h]
        num = alpha[:, h][:, None] * acc_list[h] + beta[:, h][:, None] * acc_b
        out_ref[:, 0, h, :] = num / lsum[:, h][:, None]

    pl.semaphore_signal(
        exit_sem, inc=1, device_id=nbr, device_id_type=pl.DeviceIdType.MESH
    )
    pl.semaphore_wait(exit_sem, 1)


def kernel(Q, K, V, bt, lens):
    lens2 = lens.reshape(B, 1)
    return pl.pallas_call(
        _body,
        out_shape=jax.ShapeDtypeStruct((B, 1, H, D), jnp.float32),
        in_specs=[
            pl.BlockSpec(memory_space=pltpu.VMEM),
            pl.BlockSpec(memory_space=pl.ANY),
            pl.BlockSpec(memory_space=pl.ANY),
            pl.BlockSpec(memory_space=pltpu.VMEM),
            pl.BlockSpec(memory_space=pltpu.VMEM),
        ],
        out_specs=pl.BlockSpec(memory_space=pltpu.VMEM),
        scratch_shapes=[
            pltpu.VMEM((2, LP, BS, D), jnp.float32),
            pltpu.VMEM((2, LP, BS, D), jnp.float32),
            pltpu.VMEM((H, B, D), jnp.float32),
            pltpu.VMEM((H, B, D), jnp.float32),
            pltpu.VMEM((B, 2 * H), jnp.float32),
            pltpu.VMEM((B, 2 * H), jnp.float32),
            pltpu.SemaphoreType.DMA((2,)),
            pltpu.SemaphoreType.DMA((2,)),
            pltpu.SemaphoreType.DMA((H,)),
            pltpu.SemaphoreType.DMA((H,)),
            pltpu.SemaphoreType.DMA,
            pltpu.SemaphoreType.DMA,
            pltpu.SemaphoreType.REGULAR,
        ],
        compiler_params=pltpu.CompilerParams(collective_id=0),
    )(Q, K, V, bt, lens2)


# device time: 27415 ns/iter; 3.3657x vs baseline; 2.1793x over previous
import jax
import jax.numpy as jnp
from jax import lax
from jax.experimental import pallas as pl
from jax.experimental.pallas import tpu as pltpu

B = 16
H = 16
D = 64
BS = 16
NB = 128
LP = 128
NK = LP * BS
SCALE = D ** -0.5
NEG = -1e30


def _body(q_ref, k_ref, v_ref, bt_ref, lens_ref, out_ref,
          send_acc, recv_acc, send_ml, recv_ml,
          acc_send_sem, acc_recv_sem, ml_send_sem, ml_recv_sem,
          exit_sem):
    my_x = lax.axis_index("x")
    my_y = lax.axis_index("y")
    my_z = lax.axis_index("z")
    nbr = (1 - my_x, my_y, my_z)

    with jax.named_scope("barrier"):
        barrier_sem = pltpu.get_barrier_semaphore()
        pl.semaphore_signal(
            barrier_sem, inc=1, device_id=nbr,
            device_id_type=pl.DeviceIdType.MESH,
        )
        pl.semaphore_wait(barrier_sem, 1)

    with jax.named_scope("counts"):
        slot_iota = lax.broadcasted_iota(jnp.int32, (B, NB), 1)
        valid = slot_iota < lens_ref[:, :]
        bt = jnp.where(valid, bt_ref[:, :], -1)
        page0 = my_x * LP
        pid = page0 + lax.broadcasted_iota(jnp.int32, (LP, 1), 0)
        counts = []
        for i in range(B):
            bti = bt[i, :].reshape(1, NB)
            eq = (bti == pid).astype(jnp.float32)
            counts.append(jnp.sum(eq, axis=1))
        C = jnp.stack(counts, axis=0)
        C3 = C[:, None, :]
        sel3 = C3 > 0.0

    q = q_ref[:, 0, :, :]
    m_list, l_list, acc_list, rdmas = [], [], [], []
    for h in range(H):
        with jax.named_scope(f"head{h}_compute"):
            qh = (q[:, h, :] * SCALE).astype(jnp.bfloat16)
            kh = jnp.concatenate([k_ref[t, h] for t in range(BS)], axis=1)
            s = lax.dot_general(
                qh, kh, (((1,), (0,)), ((), ())),
                preferred_element_type=jnp.float32,
            )
            s3 = s.reshape(B, BS, LP)
            sm = jnp.where(sel3, s3, NEG)
            mh = jnp.max(sm, axis=(1, 2), keepdims=True)
            p3 = jnp.where(sel3, C3 * jnp.exp(s3 - mh), 0.0)
            lh = jnp.sum(p3, axis=(1, 2))
            pb = p3.astype(jnp.bfloat16)
            acc = sum(
                lax.dot_general(
                    pb[:, t, :], v_ref[t, h], (((1,), (1,)), ((), ())),
                    preferred_element_type=jnp.float32,
                )
                for t in range(BS)
            )
        with jax.named_scope(f"head{h}_rdma"):
            send_acc[h] = acc
            rdma = pltpu.make_async_remote_copy(
                src_ref=send_acc.at[h],
                dst_ref=recv_acc.at[h],
                send_sem=acc_send_sem.at[h],
                recv_sem=acc_recv_sem.at[h],
                device_id=nbr,
                device_id_type=pl.DeviceIdType.MESH,
            )
            rdma.start()
        rdmas.append(rdma)
        m_list.append(mh.reshape(B, 1))
        l_list.append(lh.reshape(B, 1))
        acc_list.append(acc)
    with jax.named_scope("ml_send"):
        m_a = jnp.concatenate(m_list, axis=1)
        l_a = jnp.concatenate(l_list, axis=1)
        send_ml[:, 0:H] = m_a
        send_ml[:, H:2 * H] = l_a
        ml_rdma = pltpu.make_async_remote_copy(
            src_ref=send_ml,
            dst_ref=recv_ml,
            send_sem=ml_send_sem,
            recv_sem=ml_recv_sem,
            device_id=nbr,
            device_id_type=pl.DeviceIdType.MESH,
        )
        ml_rdma.start()

    with jax.named_scope("ml_wait"):
        ml_rdma.wait()
    with jax.named_scope("merge"):
        m_b = recv_ml[:, 0:H]
        l_b = recv_ml[:, H:2 * H]
        m = jnp.maximum(m_a, m_b)
        alpha = jnp.exp(m_a - m)
        beta = jnp.exp(m_b - m)
        lsum = alpha * l_a + beta * l_b
        for h in range(H):
            rdmas[h].wait()
            acc_b = recv_acc[h]
            num = alpha[:, h][:, None] * acc_list[h] + beta[:, h][:, None] * acc_b
            out_ref[:, 0, h, :] = num / lsum[:, h][:, None]

    with jax.named_scope("exit_hs"):
        pl.semaphore_signal(
            exit_sem, inc=1, device_id=nbr, device_id_type=pl.DeviceIdType.MESH
        )
        pl.semaphore_wait(exit_sem, 1)


def kernel(Q, K, V, bt, lens):
    lens2 = lens.reshape(B, 1)
    Kb = jnp.transpose(K, (1, 2, 3, 0)).astype(jnp.bfloat16)
    Vb = jnp.transpose(V, (1, 2, 3, 0)).astype(jnp.bfloat16)
    return pl.pallas_call(
        _body,
        out_shape=jax.ShapeDtypeStruct((B, 1, H, D), jnp.float32),
        in_specs=[
            pl.BlockSpec(memory_space=pltpu.VMEM),
            pl.BlockSpec(memory_space=pltpu.VMEM),
            pl.BlockSpec(memory_space=pltpu.VMEM),
            pl.BlockSpec(memory_space=pltpu.VMEM),
            pl.BlockSpec(memory_space=pltpu.VMEM),
        ],
        out_specs=pl.BlockSpec(memory_space=pltpu.VMEM),
        scratch_shapes=[
            pltpu.VMEM((H, B, D), jnp.float32),
            pltpu.VMEM((H, B, D), jnp.float32),
            pltpu.VMEM((B, 2 * H), jnp.float32),
            pltpu.VMEM((B, 2 * H), jnp.float32),
            pltpu.SemaphoreType.DMA((H,)),
            pltpu.SemaphoreType.DMA((H,)),
            pltpu.SemaphoreType.DMA,
            pltpu.SemaphoreType.DMA,
            pltpu.SemaphoreType.REGULAR,
        ],
        compiler_params=pltpu.CompilerParams(collective_id=0),
    )(Q, Kb, Vb, bt, lens2)
